# baseline (speedup 1.0000x reference)
import functools

import jax
import jax.numpy as jnp
from jax import lax
from jax.experimental import pallas as pl
from jax.experimental.pallas import tpu as pltpu
from jax.experimental.pallas import tpu_sc as plsc

N_NODES = 50000
N_EDGES = 800000

NC = 2
NS = 16
NW = NC * NS
K = 256
SUB = K // 128
NB = 100
E_PAD = NW * NB * K
ROWS_PER_SUB = 3128
N_ACC = ROWS_PER_SUB * NS
TW = 32


def _edge_pass(table, edata, d_acc, with_count, r1off):
    mesh = plsc.VectorSubcoreMesh(core_axis_name="c", subcore_axis_name="s")

    @functools.partial(
        pl.kernel,
        out_type=jax.ShapeDtypeStruct((NC, N_ACC, d_acc), jnp.float32),
        mesh=mesh,
        compiler_params=pltpu.CompilerParams(
            use_tc_tiling_on_sc=False, needs_layout_passes=False
        ),
        scratch_types=[
            [pltpu.VMEM((8, 128), jnp.int32) for _ in range(4)],
            [pltpu.VMEM((K, TW), jnp.float32) for _ in range(2)],
            pltpu.VMEM((K, d_acc), jnp.float32),
            pltpu.VMEM_SHARED((N_ACC, d_acc), jnp.float32),
            [pltpu.SemaphoreType.DMA for _ in range(4)],
            [pltpu.SemaphoreType.DMA for _ in range(2)],
            pltpu.SemaphoreType.DMA,
        ],
    )
    def kern(table_h, edata_h, out_h, ebufs, rowsb, msg, acc, isems, gsems, ssem):
        cid = lax.axis_index("c")
        sid = lax.axis_index("s")
        wid = sid * NC + cid
        tbase = wid * NB

        zero16 = jnp.zeros((16,), jnp.float32)

        @pl.loop(0, K)
        def _zero(r):
            for c in range(d_acc // 16):
                msg[r, pl.ds(c * 16, 16)] = zero16

        init_chunks = [K] * (ROWS_PER_SUB // K)
        if ROWS_PER_SUB % K:
            init_chunks.append(ROWS_PER_SUB % K)
        base = sid * ROWS_PER_SUB
        off = 0
        for sz in init_chunks:
            pltpu.sync_copy(msg.at[pl.ds(0, sz)], acc.at[pl.ds(base + off, sz)])
            off += sz

        if with_count:
            iota16 = lax.broadcasted_iota(jnp.int32, (16,), 0)
            one0 = jnp.where(iota16 == 0, 1.0, 0.0).astype(jnp.float32)

            @pl.loop(0, K)
            def _cnt(r):
                msg[r, pl.ds(16, 16)] = one0

        plsc.subcore_barrier()

        def load_edata(b, i):
            t = tbase + jnp.minimum(b, NB - 1)
            pltpu.async_copy(edata_h.at[t], ebufs[i], isems[i])

        def wait_edata(i):
            pltpu.make_async_copy(edata_h.at[0], ebufs[i], isems[i]).wait()

        def issue_gather(i, r):
            for j in range(SUB):
                pltpu.async_copy(
                    table_h.at[ebufs[i].at[j]],
                    rowsb[r].at[pl.ds(j * 128, 128)],
                    gsems[r],
                )

        def wait_gather(r):
            for j in range(SUB):
                pltpu.make_async_copy(
                    table_h.at[pl.ds(0, 128)],
                    rowsb[r].at[pl.ds(j * 128, 128)],
                    gsems[r],
                ).wait()

        def issue_scatter(i):
            for j in range(SUB):
                pltpu.async_copy(
                    msg.at[pl.ds(j * 128, 128)],
                    acc.at[ebufs[i].at[2 + j]],
                    ssem,
                    add=True,
                )

        def wait_scatter(i):
            for j in range(SUB):
                pltpu.make_async_copy(
                    msg.at[pl.ds(j * 128, 128)],
                    acc.at[ebufs[i].at[2 + j]],
                    ssem,
                ).wait()

        def compute(i, r):
            for ru in range(K // 128):
                @pl.loop(0, 8)
                def _grp(gg, ru=ru, i=i, r=r):
                    gbase = ru * 128 + gg * 16
                    ui = ebufs[i][4 + ru, pl.ds(gg * 16, 16)]
                    uf = plsc.bitcast(ui, jnp.float32)
                    for t in range(16):
                        row = gbase + t
                        ub = jnp.full((16,), uf[t], dtype=jnp.float32)
                        r0 = rowsb[r][row, pl.ds(0, 16)]
                        r1 = rowsb[r][row, pl.ds(r1off, 16)]
                        msg[row, pl.ds(0, 16)] = r0 + ub * r1

        def step(b, k, first=False):
            p2, nx4, nn4 = k % 2, (k + 1) % 4, (k + 2) % 4
            wait_gather(p2)
            if not first:
                wait_scatter((k - 1) % 4)
            wait_edata(nx4)
            issue_gather(nx4, 1 - p2)
            load_edata(b + 2, nn4)
            compute(k, p2)
            issue_scatter(k)

        load_edata(0, 0)
        wait_edata(0)
        issue_gather(0, 0)
        load_edata(1, 1)
        step(0, 0, first=True)
        step(1, 1)
        step(2, 2)
        step(3, 3)

        @pl.loop(0, (NB - 4) // 4)
        def _steady(t):
            for k in range(4):
                step(4 + 4 * t + k, k)

        wait_gather(NB % 2)
        wait_scatter((NB - 1) % 4)
        wait_edata((NB + 1) % 4)

        plsc.subcore_barrier()
        off = 0
        for sz in init_chunks:
            pltpu.sync_copy(
                acc.at[pl.ds(base + off, sz)],
                out_h.at[cid].at[pl.ds(base + off, sz)],
            )
            off += sz

    return kern(table, edata)


def _tc1(x, b1mat, bias1):
    R, G = 1000, 50

    def body(x_ref, w_ref, bias_ref, yc_ref, xr_ref):
        t = jnp.dot(x_ref[...], w_ref[...], preferred_element_type=jnp.float32)
        t = t + bias_ref[...]
        yc_ref[...] = t[:, :TW]
        xr_ref[...] = t[:, TW:]

    return pl.pallas_call(
        body,
        grid=(G,),
        in_specs=[
            pl.BlockSpec((R, 61), lambda i: (i, 0)),
            pl.BlockSpec((61, 48), lambda i: (0, 0)),
            pl.BlockSpec((1, 48), lambda i: (0, 0)),
        ],
        out_specs=[
            pl.BlockSpec((R, TW), lambda i: (i, 0)),
            pl.BlockSpec((R, 16), lambda i: (i, 0)),
        ],
        out_shape=[
            jax.ShapeDtypeStruct((N_NODES, TW), jnp.float32),
            jax.ShapeDtypeStruct((N_NODES, 16), jnp.float32),
        ],
    )(x, b1mat, bias1)


def _tc2(acc1, xr, b2mat, bias2):
    R, G = 1000, 50

    def body(a0_ref, a1_ref, xr_ref, w_ref, bias_ref, zt_ref):
        s = a0_ref[0] + a1_ref[0]
        cnt = s[:, 16:17]
        agg = s[:, :16] / jnp.maximum(cnt, 1.0)
        h = agg + xr_ref[...]
        h = jnp.where(h > 0, h, jnp.exp(h) - 1.0)
        t = jnp.dot(h, w_ref[...], preferred_element_type=jnp.float32)
        t = t + bias_ref[...]
        lane = lax.broadcasted_iota(jnp.int32, (1, TW), 1)
        zt_ref[...] = t + cnt * jnp.where(lane == 6, 1.0, 0.0).astype(jnp.float32)

    return pl.pallas_call(
        body,
        grid=(G,),
        in_specs=[
            pl.BlockSpec((1, R, TW), lambda i: (0, i, 0)),
            pl.BlockSpec((1, R, TW), lambda i: (1, i, 0)),
            pl.BlockSpec((R, 16), lambda i: (i, 0)),
            pl.BlockSpec((16, TW), lambda i: (0, 0)),
            pl.BlockSpec((1, TW), lambda i: (0, 0)),
        ],
        out_specs=pl.BlockSpec((R, TW), lambda i: (i, 0)),
        out_shape=jax.ShapeDtypeStruct((N_NODES, TW), jnp.float32),
    )(acc1, acc1, xr, b2mat, bias2)


def _tc3(acc2, zt):
    R, G = 1000, 50

    def body(a0_ref, a1_ref, zt_ref, o_ref):
        zt = zt_ref[...]
        cnt = zt[:, 6:7]
        s = a0_ref[0] + a1_ref[0]
        o = s[:, 0:2] / jnp.maximum(cnt, 1.0) + zt[:, 4:6]
        m = jnp.max(o, axis=1, keepdims=True)
        lse = m + jnp.log(jnp.sum(jnp.exp(o - m), axis=1, keepdims=True))
        o_ref[...] = o - lse

    return pl.pallas_call(
        body,
        grid=(G,),
        in_specs=[
            pl.BlockSpec((1, R, 16), lambda i: (0, i, 0)),
            pl.BlockSpec((1, R, 16), lambda i: (1, i, 0)),
            pl.BlockSpec((R, TW), lambda i: (i, 0)),
        ],
        out_specs=pl.BlockSpec((R, 2), lambda i: (i, 0)),
        out_shape=jax.ShapeDtypeStruct((N_NODES, 2), jnp.float32),
    )(acc2, acc2, zt)


def kernel(x, edge_index, edge_attr, dropout, W1, root1, b1, W2, root2, b2):
    src = edge_index[0]
    dst = edge_index[1]
    u = edge_attr[:, 0]

    pad = E_PAD - N_EDGES
    nbt = NW * NB
    src2 = jnp.concatenate([src, jnp.zeros((pad,), jnp.int32)]).reshape(nbt, SUB, 128)
    dump = N_NODES + jnp.arange(pad, dtype=jnp.int32) % (N_ACC - N_NODES)
    dst2 = jnp.concatenate([dst, dump]).reshape(nbt, SUB, 128)
    u2 = lax.bitcast_convert_type(
        jnp.concatenate([u, jnp.zeros((pad,), jnp.float32)]), jnp.int32
    ).reshape(nbt, SUB, 128)
    edata = jnp.concatenate(
        [src2, dst2, u2, jnp.zeros((nbt, 8 - 3 * SUB, 128), jnp.int32)], axis=1
    )

    b1mat = jnp.concatenate([W1[0], W1[1] - W1[0], root1], axis=1)
    bias1 = jnp.concatenate([jnp.zeros((TW,), jnp.float32), b1])[None, :]

    yc, xr = _tc1(x, b1mat, bias1)
    acc1 = _edge_pass(yc, edata, d_acc=TW, with_count=True, r1off=16)

    b2mat = (
        jnp.zeros((16, TW), jnp.float32)
        .at[:, 0:2].set(W2[0])
        .at[:, 8:10].set(W2[1] - W2[0])
        .at[:, 4:6].set(root2)
    )
    bias2 = jnp.zeros((1, TW), jnp.float32).at[0, 4:6].set(b2)

    zt = _tc2(acc1, xr, b2mat, bias2)
    acc2 = _edge_pass(zt, edata, d_acc=16, with_count=False, r1off=8)
    return _tc3(acc2, zt)


# speedup vs baseline: 1.4823x; 1.4823x over previous
import functools

import jax
import jax.numpy as jnp
from jax import lax
from jax.experimental import pallas as pl
from jax.experimental.pallas import tpu as pltpu
from jax.experimental.pallas import tpu_sc as plsc

N_NODES = 50000
N_EDGES = 800000

NC = 2
NS = 16
NW = NC * NS
K = 256
SUB = K // 128
NB = 100
E_PAD = NW * NB * K
ROWS_PER_SUB = 3128
N_ACC = ROWS_PER_SUB * NS
TW = 32


def _edge_pass(table, edata, d_acc, with_count, r1off):
    mesh = plsc.VectorSubcoreMesh(core_axis_name="c", subcore_axis_name="s")

    @functools.partial(
        pl.kernel,
        out_type=jax.ShapeDtypeStruct((NC, N_ACC, d_acc), jnp.float32),
        mesh=mesh,
        compiler_params=pltpu.CompilerParams(
            use_tc_tiling_on_sc=False,
            needs_layout_passes=False,
            disable_bounds_checks=True,
        ),
        scratch_types=[
            [pltpu.VMEM((8, 128), jnp.int32) for _ in range(4)],
            [pltpu.VMEM((K, TW), jnp.float32) for _ in range(2)],
            pltpu.VMEM((K, d_acc), jnp.float32),
            pltpu.VMEM_SHARED((N_ACC, d_acc), jnp.float32),
            [pltpu.SemaphoreType.DMA for _ in range(4)],
            [pltpu.SemaphoreType.DMA for _ in range(2)],
            pltpu.SemaphoreType.DMA,
        ],
    )
    def kern(table_h, edata_h, out_h, ebufs, rowsb, msg, acc, isems, gsems, ssem):
        cid = lax.axis_index("c")
        sid = lax.axis_index("s")
        wid = sid * NC + cid
        tbase = wid * NB

        zero16 = jnp.zeros((16,), jnp.float32)

        @pl.loop(0, K)
        def _zero(r):
            for c in range(d_acc // 16):
                msg[r, pl.ds(c * 16, 16)] = zero16

        init_chunks = [K] * (ROWS_PER_SUB // K)
        if ROWS_PER_SUB % K:
            init_chunks.append(ROWS_PER_SUB % K)
        base = sid * ROWS_PER_SUB
        off = 0
        for sz in init_chunks:
            pltpu.sync_copy(msg.at[pl.ds(0, sz)], acc.at[pl.ds(base + off, sz)])
            off += sz

        if with_count:
            iota16 = lax.broadcasted_iota(jnp.int32, (16,), 0)
            one0 = jnp.where(iota16 == 0, 1.0, 0.0).astype(jnp.float32)

            @pl.loop(0, K)
            def _cnt(r):
                msg[r, pl.ds(16, 16)] = one0

        plsc.subcore_barrier()

        def load_edata(b, i):
            t = tbase + jnp.minimum(b, NB - 1)
            pltpu.async_copy(edata_h.at[t], ebufs[i], isems[i])

        def wait_edata(i):
            pltpu.make_async_copy(edata_h.at[0], ebufs[i], isems[i]).wait()

        def issue_gather(i, r):
            for j in range(SUB):
                pltpu.async_copy(
                    table_h.at[ebufs[i].at[j]],
                    rowsb[r].at[pl.ds(j * 128, 128)],
                    gsems[r],
                )

        def wait_gather(r):
            for j in range(SUB):
                pltpu.make_async_copy(
                    table_h.at[pl.ds(0, 128)],
                    rowsb[r].at[pl.ds(j * 128, 128)],
                    gsems[r],
                ).wait()

        def issue_scatter(i):
            for j in range(SUB):
                pltpu.async_copy(
                    msg.at[pl.ds(j * 128, 128)],
                    acc.at[ebufs[i].at[2 + j]],
                    ssem,
                    add=True,
                )

        def wait_scatter(i):
            for j in range(SUB):
                pltpu.make_async_copy(
                    msg.at[pl.ds(j * 128, 128)],
                    acc.at[ebufs[i].at[2 + j]],
                    ssem,
                ).wait()

        def compute(i, r):
            for ru in range(K // 128):
                @plsc.parallel_loop(0, 8, unroll=2)
                def _grp(gg, ru=ru, i=i, r=r):
                    gbase = ru * 128 + gg * 16
                    ui = ebufs[i][4 + ru, pl.ds(gg * 16, 16)]
                    uf = plsc.bitcast(ui, jnp.float32)
                    for t in range(16):
                        row = gbase + t
                        ub = jnp.full((16,), uf[t], dtype=jnp.float32)
                        r0 = rowsb[r][row, pl.ds(0, 16)]
                        r1 = rowsb[r][row, pl.ds(r1off, 16)]
                        msg[row, pl.ds(0, 16)] = r0 + ub * r1

        def step(b, k, first=False):
            p2, nx4, nn4 = k % 2, (k + 1) % 4, (k + 2) % 4
            wait_gather(p2)
            if not first:
                wait_scatter((k - 1) % 4)
            wait_edata(nx4)
            issue_gather(nx4, 1 - p2)
            load_edata(b + 2, nn4)
            compute(k, p2)
            issue_scatter(k)

        load_edata(0, 0)
        wait_edata(0)
        issue_gather(0, 0)
        load_edata(1, 1)
        step(0, 0, first=True)
        step(1, 1)
        step(2, 2)
        step(3, 3)

        @pl.loop(0, (NB - 4) // 4)
        def _steady(t):
            for k in range(4):
                step(4 + 4 * t + k, k)

        wait_gather(NB % 2)
        wait_scatter((NB - 1) % 4)
        wait_edata((NB + 1) % 4)

        plsc.subcore_barrier()
        off = 0
        for sz in init_chunks:
            pltpu.sync_copy(
                acc.at[pl.ds(base + off, sz)],
                out_h.at[cid].at[pl.ds(base + off, sz)],
            )
            off += sz

    return kern(table, edata)


def _tc1(x, b1mat, bias1):
    R, G = 1000, 50

    def body(x_ref, w_ref, bias_ref, yc_ref, xr_ref):
        t = jnp.dot(x_ref[...], w_ref[...], preferred_element_type=jnp.float32)
        t = t + bias_ref[...]
        yc_ref[...] = t[:, :TW]
        xr_ref[...] = t[:, TW:]

    return pl.pallas_call(
        body,
        grid=(G,),
        in_specs=[
            pl.BlockSpec((R, 61), lambda i: (i, 0)),
            pl.BlockSpec((61, 48), lambda i: (0, 0)),
            pl.BlockSpec((1, 48), lambda i: (0, 0)),
        ],
        out_specs=[
            pl.BlockSpec((R, TW), lambda i: (i, 0)),
            pl.BlockSpec((R, 16), lambda i: (i, 0)),
        ],
        out_shape=[
            jax.ShapeDtypeStruct((N_NODES, TW), jnp.float32),
            jax.ShapeDtypeStruct((N_NODES, 16), jnp.float32),
        ],
    )(x, b1mat, bias1)


def _tc2(acc1, xr, b2mat, bias2):
    R, G = 1000, 50

    def body(a0_ref, a1_ref, xr_ref, w_ref, bias_ref, zt_ref):
        s = a0_ref[0] + a1_ref[0]
        cnt = s[:, 16:17]
        agg = s[:, :16] / jnp.maximum(cnt, 1.0)
        h = agg + xr_ref[...]
        h = jnp.where(h > 0, h, jnp.exp(h) - 1.0)
        t = jnp.dot(h, w_ref[...], preferred_element_type=jnp.float32)
        t = t + bias_ref[...]
        lane = lax.broadcasted_iota(jnp.int32, (1, TW), 1)
        zt_ref[...] = t + cnt * jnp.where(lane == 6, 1.0, 0.0).astype(jnp.float32)

    return pl.pallas_call(
        body,
        grid=(G,),
        in_specs=[
            pl.BlockSpec((1, R, TW), lambda i: (0, i, 0)),
            pl.BlockSpec((1, R, TW), lambda i: (1, i, 0)),
            pl.BlockSpec((R, 16), lambda i: (i, 0)),
            pl.BlockSpec((16, TW), lambda i: (0, 0)),
            pl.BlockSpec((1, TW), lambda i: (0, 0)),
        ],
        out_specs=pl.BlockSpec((R, TW), lambda i: (i, 0)),
        out_shape=jax.ShapeDtypeStruct((N_NODES, TW), jnp.float32),
    )(acc1, acc1, xr, b2mat, bias2)


def _tc3(acc2, zt):
    R, G = 1000, 50

    def body(a0_ref, a1_ref, zt_ref, o_ref):
        zt = zt_ref[...]
        cnt = zt[:, 6:7]
        s = a0_ref[0] + a1_ref[0]
        o = s[:, 0:2] / jnp.maximum(cnt, 1.0) + zt[:, 4:6]
        m = jnp.max(o, axis=1, keepdims=True)
        lse = m + jnp.log(jnp.sum(jnp.exp(o - m), axis=1, keepdims=True))
        o_ref[...] = o - lse

    return pl.pallas_call(
        body,
        grid=(G,),
        in_specs=[
            pl.BlockSpec((1, R, 16), lambda i: (0, i, 0)),
            pl.BlockSpec((1, R, 16), lambda i: (1, i, 0)),
            pl.BlockSpec((R, TW), lambda i: (i, 0)),
        ],
        out_specs=pl.BlockSpec((R, 2), lambda i: (i, 0)),
        out_shape=jax.ShapeDtypeStruct((N_NODES, 2), jnp.float32),
    )(acc2, acc2, zt)


def kernel(x, edge_index, edge_attr, dropout, W1, root1, b1, W2, root2, b2):
    src = edge_index[0]
    dst = edge_index[1]
    u = edge_attr[:, 0]

    pad = E_PAD - N_EDGES
    nbt = NW * NB
    psrc = jnp.arange(pad, dtype=jnp.int32) * 61 % N_NODES
    src2 = jnp.concatenate([src, psrc]).reshape(nbt, SUB, 128)
    dump = N_NODES + jnp.arange(pad, dtype=jnp.int32) % (N_ACC - N_NODES)
    dst2 = jnp.concatenate([dst, dump]).reshape(nbt, SUB, 128)
    u2 = lax.bitcast_convert_type(
        jnp.concatenate([u, jnp.zeros((pad,), jnp.float32)]), jnp.int32
    ).reshape(nbt, SUB, 128)
    edata = jnp.concatenate(
        [src2, dst2, u2, jnp.zeros((nbt, 8 - 3 * SUB, 128), jnp.int32)], axis=1
    )

    b1mat = jnp.concatenate([W1[0], W1[1] - W1[0], root1], axis=1)
    bias1 = jnp.concatenate([jnp.zeros((TW,), jnp.float32), b1])[None, :]

    yc, xr = _tc1(x, b1mat, bias1)
    acc1 = _edge_pass(yc, edata, d_acc=TW, with_count=True, r1off=16)

    b2mat = (
        jnp.zeros((16, TW), jnp.float32)
        .at[:, 0:2].set(W2[0])
        .at[:, 8:10].set(W2[1] - W2[0])
        .at[:, 4:6].set(root2)
    )
    bias2 = jnp.zeros((1, TW), jnp.float32).at[0, 4:6].set(b2)

    zt = _tc2(acc1, xr, b2mat, bias2)
    acc2 = _edge_pass(zt, edata, d_acc=16, with_count=False, r1off=8)
    return _tc3(acc2, zt)
